# baseline (speedup 1.0000x reference)
import functools

import jax
import jax.numpy as jnp
from jax.experimental import pallas as pl
from jax.experimental.pallas import tpu as pltpu

KK = 7
NN = 32768
BB = 64
SS = NN // BB
EG = SS * KK
EPS = 1e-5


def _silu(a):
    return a * (1.0 / (1.0 + jnp.exp(-a)))


def _norm_silu(h, mean, sv, gam, bet):
    return _silu((h - mean) / sv * gam + bet)


def _dot(a, b):
    return jax.lax.dot_general(a, b, (((1,), (0,)), ((), ())),
                               preferred_element_type=jnp.float32)


def _dot_exact(a, b):
    return jax.lax.dot_general(a, b, (((1,), (0,)), ((), ())),
                               preferred_element_type=jnp.float32,
                               precision=jax.lax.Precision.HIGHEST)


def _knnl1_body(pos_ref, x_ref, w1_ref, b1_ref, wn_ref, bn_ref,
                t_ref, u_ref):
    posg = pos_ref[0]
    pp = posg * posg
    sqc = jnp.sum(pp, axis=1, keepdims=True)
    gram = jax.lax.dot_general(posg, posg, (((1,), (1,)), ((), ())),
                               preferred_element_type=jnp.float32)
    rows = jax.lax.broadcasted_iota(jnp.int32, (SS, SS), 0)
    cols = jax.lax.broadcasted_iota(jnp.int32, (SS, SS), 1)
    score = (sqc + sqc.T) - 2.0 * gram
    score = score + jnp.where(rows == cols, jnp.float32(1e10), 0.0)

    xg = x_ref[0]
    tks = []
    for k in range(KK):
        v = jnp.min(score, axis=1, keepdims=True)
        cand = jnp.where(score == v, cols, SS)
        am = jnp.min(cand, axis=1, keepdims=True)
        sel = cols == am
        oh = jnp.where(sel, 1.0, 0.0).astype(jnp.float32)
        xj = _dot_exact(oh, xg)
        e = jnp.concatenate([xg, xj - xg], axis=1)
        tks.append((_dot(e, w1_ref[...]) + b1_ref[...])[None])
        if k < KK - 1:
            score = jnp.where(sel, jnp.float32(2e30), score)
    tkm = jnp.concatenate(tks, axis=0)
    t_ref[0] = jnp.transpose(tkm, (1, 0, 2)).reshape(EG, -1)
    u_ref[0] = _dot(xg, wn_ref[...]) + bn_ref[...]


def _knn_layer1(pos3d, x3d, w1, b1, wn, bn):
    dp = pos3d.shape[-1]
    dx = x3d.shape[-1]
    c = w1.shape[1]
    return pl.pallas_call(
        _knnl1_body,
        grid=(BB,),
        in_specs=[
            pl.BlockSpec((1, SS, dp), lambda g: (g, 0, 0)),
            pl.BlockSpec((1, SS, dx), lambda g: (g, 0, 0)),
            pl.BlockSpec((2 * dx, c), lambda g: (0, 0)),
            pl.BlockSpec((1, c), lambda g: (0, 0)),
            pl.BlockSpec((dx, c), lambda g: (0, 0)),
            pl.BlockSpec((1, c), lambda g: (0, 0)),
        ],
        out_specs=[
            pl.BlockSpec((1, EG, c), lambda g: (g, 0, 0)),
            pl.BlockSpec((1, SS, c), lambda g: (g, 0, 0)),
        ],
        out_shape=[
            jax.ShapeDtypeStruct((BB, EG, c), jnp.float32),
            jax.ShapeDtypeStruct((BB, SS, c), jnp.float32),
        ],
    )(pos3d, x3d, w1, b1, wn, bn)


def _mid_body(t_ref, u_ref, me_ref, se_ref, ge_ref, te_ref,
              mn_ref, sn_ref, gn_ref, tn_ref,
              we_ref, be_ref, wn_ref, bn_ref, to_ref, uo_ref):
    ae = _norm_silu(t_ref[0], me_ref[...], se_ref[...], ge_ref[...],
                    te_ref[...])
    to_ref[0] = _dot(ae, we_ref[...]) + be_ref[...]
    an = _norm_silu(u_ref[0], mn_ref[...], sn_ref[...], gn_ref[...],
                    tn_ref[...])
    uo_ref[0] = _dot(an, wn_ref[...]) + bn_ref[...]


def _midlayer(t, u, st_e, st_n, we, be, wn, bn):
    cin, c = we.shape
    vec = pl.BlockSpec((1, cin), lambda g: (0, 0))
    return pl.pallas_call(
        _mid_body,
        grid=(BB,),
        in_specs=[
            pl.BlockSpec((1, EG, cin), lambda g: (g, 0, 0)),
            pl.BlockSpec((1, SS, cin), lambda g: (g, 0, 0)),
            vec, vec, vec, vec, vec, vec, vec, vec,
            pl.BlockSpec((cin, c), lambda g: (0, 0)),
            pl.BlockSpec((1, c), lambda g: (0, 0)),
            pl.BlockSpec((cin, c), lambda g: (0, 0)),
            pl.BlockSpec((1, c), lambda g: (0, 0)),
        ],
        out_specs=[
            pl.BlockSpec((1, EG, c), lambda g: (g, 0, 0)),
            pl.BlockSpec((1, SS, c), lambda g: (g, 0, 0)),
        ],
        out_shape=[
            jax.ShapeDtypeStruct((BB, EG, c), jnp.float32),
            jax.ShapeDtypeStruct((BB, SS, c), jnp.float32),
        ],
    )(t, u, *st_e, *st_n, we, be, wn, bn)


def _comb_body(t_ref, u_ref, me_ref, se_ref, ge_ref, te_ref,
               mn_ref, sn_ref, gn_ref, tn_ref, h_ref):
    ae = _norm_silu(t_ref[0], me_ref[...], se_ref[...], ge_ref[...],
                    te_ref[...])
    c = ae.shape[1]
    m = jnp.max(ae.reshape(SS, KK, c), axis=1)
    an = _norm_silu(u_ref[0], mn_ref[...], sn_ref[...], gn_ref[...],
                    tn_ref[...])
    h_ref[0] = m + an


def _combine(t, u, st_e, st_n):
    c = t.shape[-1]
    vec = pl.BlockSpec((1, c), lambda g: (0, 0))
    return pl.pallas_call(
        _comb_body,
        grid=(BB,),
        in_specs=[
            pl.BlockSpec((1, EG, c), lambda g: (g, 0, 0)),
            pl.BlockSpec((1, SS, c), lambda g: (g, 0, 0)),
            vec, vec, vec, vec, vec, vec, vec, vec,
        ],
        out_specs=pl.BlockSpec((1, SS, c), lambda g: (g, 0, 0)),
        out_shape=jax.ShapeDtypeStruct((BB, SS, c), jnp.float32),
    )(t, u, *st_e, *st_n)


def _final_body(t_ref, u_ref, me_ref, se_ref, ge_ref, te_ref,
                mn_ref, sn_ref, gn_ref, tn_ref, wo_ref, bo_ref, p_ref):
    g = pl.program_id(0)
    ae = _norm_silu(t_ref[0], me_ref[...], se_ref[...], ge_ref[...],
                    te_ref[...])
    c = ae.shape[1]
    m = jnp.max(ae.reshape(SS, KK, c), axis=1)
    an = _norm_silu(u_ref[0], mn_ref[...], sn_ref[...], gn_ref[...],
                    tn_ref[...])
    h = m + an
    pooled = jnp.sum(h, axis=0, keepdims=True) * (1.0 / SS)
    p_ref[pl.ds(g, 1), :] = _dot(pooled, wo_ref[...]) + bo_ref[...]


def _finalize(t, u, st_e, st_n, w_out, b_out):
    c = t.shape[-1]
    vec = pl.BlockSpec((1, c), lambda g: (0, 0))
    return pl.pallas_call(
        _final_body,
        grid=(BB,),
        in_specs=[
            pl.BlockSpec((1, EG, c), lambda g: (g, 0, 0)),
            pl.BlockSpec((1, SS, c), lambda g: (g, 0, 0)),
            vec, vec, vec, vec, vec, vec, vec, vec,
            pl.BlockSpec((c, 1), lambda g: (0, 0)),
            pl.BlockSpec((1, 1), lambda g: (0, 0)),
        ],
        out_specs=pl.BlockSpec((BB, 1), lambda g: (0, 0)),
        out_shape=jax.ShapeDtypeStruct((BB, 1), jnp.float32),
    )(t, u, *st_e, *st_n, w_out, b_out)


def _bn_stats(arr3d, gamma, beta):
    flat = arr3d.reshape(-1, arr3d.shape[-1])
    mean = jnp.mean(flat, axis=0, keepdims=True)
    var = jnp.var(flat, axis=0, keepdims=True)
    sv = jnp.sqrt(var + EPS)
    return (mean, sv, gamma.reshape(1, -1), beta.reshape(1, -1))


def _dyn_conv(x3d, pos3d, ec, nnp):
    (w1, b1, g1, t1p), (w2, b2, g2, t2p), (w3, b3, g3, t3p) = ec
    (nw1, nb1, ng1, nt1), (nw2, nb2, ng2, nt2), (nw3, nb3, ng3, nt3) = nnp
    t, u = _knn_layer1(pos3d, x3d, w1, b1.reshape(1, -1),
                       nw1, nb1.reshape(1, -1))
    ste, stn = _bn_stats(t, g1, t1p), _bn_stats(u, ng1, nt1)
    t, u = _midlayer(t, u, ste, stn, w2, b2.reshape(1, -1),
                     nw2, nb2.reshape(1, -1))
    ste, stn = _bn_stats(t, g2, t2p), _bn_stats(u, ng2, nt2)
    t, u = _midlayer(t, u, ste, stn, w3, b3.reshape(1, -1),
                     nw3, nb3.reshape(1, -1))
    ste, stn = _bn_stats(t, g3, t3p), _bn_stats(u, ng3, nt3)
    return t, u, ste, stn


def kernel(x, pos, batch, ec1, nn1, ec2, nn2, w_out, b_out):
    del batch
    x3d = x.reshape(BB, SS, -1)
    pos3d = pos.reshape(BB, SS, -1)
    t, u, ste, stn = _dyn_conv(x3d, pos3d, ec1, nn1)
    h1 = _combine(t, u, ste, stn)
    t, u, ste, stn = _dyn_conv(h1, h1, ec2, nn2)
    return _finalize(t, u, ste, stn, w_out, b_out.reshape(1, 1))


# speedup vs baseline: 1.2485x; 1.2485x over previous
import functools

import jax
import jax.numpy as jnp
from jax.experimental import pallas as pl
from jax.experimental.pallas import tpu as pltpu

KK = 7
NN = 32768
BB = 64
SS = NN // BB
EG = SS * KK
EPS = 1e-5


def _silu(a):
    return a * (1.0 / (1.0 + jnp.exp(-a)))


def _norm_silu(h, mean, sv, gam, bet):
    return _silu((h - mean) / sv * gam + bet)


def _dot(a, b):
    return jax.lax.dot_general(a, b, (((1,), (0,)), ((), ())),
                               preferred_element_type=jnp.float32)


def _knnl1_body(pos_ref, x_ref, w1_ref, b1_ref, wn_ref, bn_ref,
                t_ref, u_ref):
    posg = pos_ref[0]
    pp = posg * posg
    sqc = jnp.sum(pp, axis=1, keepdims=True)
    gram = jax.lax.dot_general(posg, posg, (((1,), (1,)), ((), ())),
                               preferred_element_type=jnp.float32)
    rows = jax.lax.broadcasted_iota(jnp.int32, (SS, SS), 0)
    cols = jax.lax.broadcasted_iota(jnp.int32, (SS, SS), 1)
    score = (sqc + sqc.T) - 2.0 * gram
    score = score + jnp.where(rows == cols, jnp.float32(1e10), 0.0)

    xg = x_ref[0]
    xhi = (xg.astype(jnp.bfloat16)).astype(jnp.float32)
    xr = xg - xhi
    xmid = (xr.astype(jnp.bfloat16)).astype(jnp.float32)
    xlo = xr - xmid
    tks = []
    for k in range(KK):
        v = jnp.min(score, axis=1, keepdims=True)
        cand = jnp.where(score == v, cols, SS)
        am = jnp.min(cand, axis=1, keepdims=True)
        sel = cols == am
        oh = jnp.where(sel, 1.0, 0.0).astype(jnp.float32)
        xj = (_dot(oh, xhi) + _dot(oh, xmid)) + _dot(oh, xlo)
        e = jnp.concatenate([xg, xj - xg], axis=1)
        tks.append((_dot(e, w1_ref[...]) + b1_ref[...])[None])
        if k < KK - 1:
            score = jnp.where(sel, jnp.float32(2e30), score)
    tkm = jnp.concatenate(tks, axis=0)
    t_ref[0] = jnp.transpose(tkm, (1, 0, 2)).reshape(EG, -1)
    u_ref[0] = _dot(xg, wn_ref[...]) + bn_ref[...]


def _knn_layer1(pos3d, x3d, w1, b1, wn, bn):
    dp = pos3d.shape[-1]
    dx = x3d.shape[-1]
    c = w1.shape[1]
    return pl.pallas_call(
        _knnl1_body,
        grid=(BB,),
        in_specs=[
            pl.BlockSpec((1, SS, dp), lambda g: (g, 0, 0)),
            pl.BlockSpec((1, SS, dx), lambda g: (g, 0, 0)),
            pl.BlockSpec((2 * dx, c), lambda g: (0, 0)),
            pl.BlockSpec((1, c), lambda g: (0, 0)),
            pl.BlockSpec((dx, c), lambda g: (0, 0)),
            pl.BlockSpec((1, c), lambda g: (0, 0)),
        ],
        out_specs=[
            pl.BlockSpec((1, EG, c), lambda g: (g, 0, 0)),
            pl.BlockSpec((1, SS, c), lambda g: (g, 0, 0)),
        ],
        out_shape=[
            jax.ShapeDtypeStruct((BB, EG, c), jnp.float32),
            jax.ShapeDtypeStruct((BB, SS, c), jnp.float32),
        ],
    )(pos3d, x3d, w1, b1, wn, bn)


def _mid_body(t_ref, u_ref, me_ref, se_ref, ge_ref, te_ref,
              mn_ref, sn_ref, gn_ref, tn_ref,
              we_ref, be_ref, wn_ref, bn_ref, to_ref, uo_ref):
    ae = _norm_silu(t_ref[0], me_ref[...], se_ref[...], ge_ref[...],
                    te_ref[...])
    to_ref[0] = _dot(ae, we_ref[...]) + be_ref[...]
    an = _norm_silu(u_ref[0], mn_ref[...], sn_ref[...], gn_ref[...],
                    tn_ref[...])
    uo_ref[0] = _dot(an, wn_ref[...]) + bn_ref[...]


def _midlayer(t, u, st_e, st_n, we, be, wn, bn):
    cin, c = we.shape
    vec = pl.BlockSpec((1, cin), lambda g: (0, 0))
    return pl.pallas_call(
        _mid_body,
        grid=(BB,),
        in_specs=[
            pl.BlockSpec((1, EG, cin), lambda g: (g, 0, 0)),
            pl.BlockSpec((1, SS, cin), lambda g: (g, 0, 0)),
            vec, vec, vec, vec, vec, vec, vec, vec,
            pl.BlockSpec((cin, c), lambda g: (0, 0)),
            pl.BlockSpec((1, c), lambda g: (0, 0)),
            pl.BlockSpec((cin, c), lambda g: (0, 0)),
            pl.BlockSpec((1, c), lambda g: (0, 0)),
        ],
        out_specs=[
            pl.BlockSpec((1, EG, c), lambda g: (g, 0, 0)),
            pl.BlockSpec((1, SS, c), lambda g: (g, 0, 0)),
        ],
        out_shape=[
            jax.ShapeDtypeStruct((BB, EG, c), jnp.float32),
            jax.ShapeDtypeStruct((BB, SS, c), jnp.float32),
        ],
    )(t, u, *st_e, *st_n, we, be, wn, bn)


def _comb_body(t_ref, u_ref, me_ref, se_ref, ge_ref, te_ref,
               mn_ref, sn_ref, gn_ref, tn_ref, h_ref):
    ae = _norm_silu(t_ref[0], me_ref[...], se_ref[...], ge_ref[...],
                    te_ref[...])
    c = ae.shape[1]
    m = jnp.max(ae.reshape(SS, KK, c), axis=1)
    an = _norm_silu(u_ref[0], mn_ref[...], sn_ref[...], gn_ref[...],
                    tn_ref[...])
    h_ref[0] = m + an


def _combine(t, u, st_e, st_n):
    c = t.shape[-1]
    vec = pl.BlockSpec((1, c), lambda g: (0, 0))
    return pl.pallas_call(
        _comb_body,
        grid=(BB,),
        in_specs=[
            pl.BlockSpec((1, EG, c), lambda g: (g, 0, 0)),
            pl.BlockSpec((1, SS, c), lambda g: (g, 0, 0)),
            vec, vec, vec, vec, vec, vec, vec, vec,
        ],
        out_specs=pl.BlockSpec((1, SS, c), lambda g: (g, 0, 0)),
        out_shape=jax.ShapeDtypeStruct((BB, SS, c), jnp.float32),
    )(t, u, *st_e, *st_n)


def _final_body(t_ref, u_ref, me_ref, se_ref, ge_ref, te_ref,
                mn_ref, sn_ref, gn_ref, tn_ref, wo_ref, bo_ref, p_ref):
    g = pl.program_id(0)
    ae = _norm_silu(t_ref[0], me_ref[...], se_ref[...], ge_ref[...],
                    te_ref[...])
    c = ae.shape[1]
    m = jnp.max(ae.reshape(SS, KK, c), axis=1)
    an = _norm_silu(u_ref[0], mn_ref[...], sn_ref[...], gn_ref[...],
                    tn_ref[...])
    h = m + an
    pooled = jnp.sum(h, axis=0, keepdims=True) * (1.0 / SS)
    p_ref[pl.ds(g, 1), :] = _dot(pooled, wo_ref[...]) + bo_ref[...]


def _finalize(t, u, st_e, st_n, w_out, b_out):
    c = t.shape[-1]
    vec = pl.BlockSpec((1, c), lambda g: (0, 0))
    return pl.pallas_call(
        _final_body,
        grid=(BB,),
        in_specs=[
            pl.BlockSpec((1, EG, c), lambda g: (g, 0, 0)),
            pl.BlockSpec((1, SS, c), lambda g: (g, 0, 0)),
            vec, vec, vec, vec, vec, vec, vec, vec,
            pl.BlockSpec((c, 1), lambda g: (0, 0)),
            pl.BlockSpec((1, 1), lambda g: (0, 0)),
        ],
        out_specs=pl.BlockSpec((BB, 1), lambda g: (0, 0)),
        out_shape=jax.ShapeDtypeStruct((BB, 1), jnp.float32),
    )(t, u, *st_e, *st_n, w_out, b_out)


def _bn_stats(arr3d, gamma, beta):
    flat = arr3d.reshape(-1, arr3d.shape[-1])
    mean = jnp.mean(flat, axis=0, keepdims=True)
    var = jnp.var(flat, axis=0, keepdims=True)
    sv = jnp.sqrt(var + EPS)
    return (mean, sv, gamma.reshape(1, -1), beta.reshape(1, -1))


def _dyn_conv(x3d, pos3d, ec, nnp):
    (w1, b1, g1, t1p), (w2, b2, g2, t2p), (w3, b3, g3, t3p) = ec
    (nw1, nb1, ng1, nt1), (nw2, nb2, ng2, nt2), (nw3, nb3, ng3, nt3) = nnp
    t, u = _knn_layer1(pos3d, x3d, w1, b1.reshape(1, -1),
                       nw1, nb1.reshape(1, -1))
    ste, stn = _bn_stats(t, g1, t1p), _bn_stats(u, ng1, nt1)
    t, u = _midlayer(t, u, ste, stn, w2, b2.reshape(1, -1),
                     nw2, nb2.reshape(1, -1))
    ste, stn = _bn_stats(t, g2, t2p), _bn_stats(u, ng2, nt2)
    t, u = _midlayer(t, u, ste, stn, w3, b3.reshape(1, -1),
                     nw3, nb3.reshape(1, -1))
    ste, stn = _bn_stats(t, g3, t3p), _bn_stats(u, ng3, nt3)
    return t, u, ste, stn


def kernel(x, pos, batch, ec1, nn1, ec2, nn2, w_out, b_out):
    del batch
    x3d = x.reshape(BB, SS, -1)
    pos3d = pos.reshape(BB, SS, -1)
    t, u, ste, stn = _dyn_conv(x3d, pos3d, ec1, nn1)
    h1 = _combine(t, u, ste, stn)
    t, u, ste, stn = _dyn_conv(h1, h1, ec2, nn2)
    return _finalize(t, u, ste, stn, w_out, b_out.reshape(1, 1))


# speedup vs baseline: 1.4661x; 1.1743x over previous
import functools

import jax
import jax.numpy as jnp
from jax.experimental import pallas as pl
from jax.experimental.pallas import tpu as pltpu

KK = 7
NN = 32768
BB = 64
SS = NN // BB
EG = SS * KK
EPS = 1e-5


def _silu(a):
    return a * (1.0 / (1.0 + jnp.exp(-a)))


def _norm_silu(h, mean, sv, gam, bet):
    return _silu((h - mean) / sv * gam + bet)


def _dot(a, b):
    return jax.lax.dot_general(a, b, (((1,), (0,)), ((), ())),
                               preferred_element_type=jnp.float32)


def _knnl1_body(pos_ref, x_ref, w1_ref, b1_ref, wn_ref, bn_ref,
                t_ref, u_ref):
    posg = pos_ref[0]
    pp = posg * posg
    sqc = jnp.sum(pp, axis=1, keepdims=True)
    gram = jax.lax.dot_general(posg, posg, (((1,), (1,)), ((), ())),
                               preferred_element_type=jnp.float32)
    rows = jax.lax.broadcasted_iota(jnp.int32, (SS, SS), 0)
    cols = jax.lax.broadcasted_iota(jnp.int32, (SS, SS), 1)
    score = (sqc + sqc.T) - 2.0 * gram
    score = score + jnp.where(rows == cols, jnp.float32(1e10), 0.0)

    xg = x_ref[0]
    xhi = (xg.astype(jnp.bfloat16)).astype(jnp.float32)
    xr = xg - xhi
    xmid = (xr.astype(jnp.bfloat16)).astype(jnp.float32)
    xlo = xr - xmid
    tks = []
    for k in range(KK):
        v = jnp.min(score, axis=1, keepdims=True)
        cand = jnp.where(score == v, cols, SS)
        am = jnp.min(cand, axis=1, keepdims=True)
        sel = cols == am
        oh = jnp.where(sel, 1.0, 0.0).astype(jnp.float32)
        xj = (_dot(oh, xhi) + _dot(oh, xmid)) + _dot(oh, xlo)
        e = jnp.concatenate([xg, xj - xg], axis=1)
        tks.append((_dot(e, w1_ref[...]) + b1_ref[...])[None])
        if k < KK - 1:
            score = jnp.where(sel, jnp.float32(2e30), score)
    t_ref[0] = jnp.concatenate(tks, axis=0).reshape(EG, -1)
    u_ref[0] = _dot(xg, wn_ref[...]) + bn_ref[...]


def _knn_layer1(pos3d, x3d, w1, b1, wn, bn):
    dp = pos3d.shape[-1]
    dx = x3d.shape[-1]
    c = w1.shape[1]
    return pl.pallas_call(
        _knnl1_body,
        grid=(BB,),
        in_specs=[
            pl.BlockSpec((1, SS, dp), lambda g: (g, 0, 0)),
            pl.BlockSpec((1, SS, dx), lambda g: (g, 0, 0)),
            pl.BlockSpec((2 * dx, c), lambda g: (0, 0)),
            pl.BlockSpec((1, c), lambda g: (0, 0)),
            pl.BlockSpec((dx, c), lambda g: (0, 0)),
            pl.BlockSpec((1, c), lambda g: (0, 0)),
        ],
        out_specs=[
            pl.BlockSpec((1, EG, c), lambda g: (g, 0, 0)),
            pl.BlockSpec((1, SS, c), lambda g: (g, 0, 0)),
        ],
        out_shape=[
            jax.ShapeDtypeStruct((BB, EG, c), jnp.float32),
            jax.ShapeDtypeStruct((BB, SS, c), jnp.float32),
        ],
    )(pos3d, x3d, w1, b1, wn, bn)


def _mid_body(t_ref, u_ref, me_ref, se_ref, ge_ref, te_ref,
              mn_ref, sn_ref, gn_ref, tn_ref,
              we_ref, be_ref, wn_ref, bn_ref, to_ref, uo_ref):
    ae = _norm_silu(t_ref[0], me_ref[...], se_ref[...], ge_ref[...],
                    te_ref[...])
    to_ref[0] = _dot(ae, we_ref[...]) + be_ref[...]
    an = _norm_silu(u_ref[0], mn_ref[...], sn_ref[...], gn_ref[...],
                    tn_ref[...])
    uo_ref[0] = _dot(an, wn_ref[...]) + bn_ref[...]


def _midlayer(t, u, st_e, st_n, we, be, wn, bn):
    cin, c = we.shape
    vec = pl.BlockSpec((1, cin), lambda g: (0, 0))
    return pl.pallas_call(
        _mid_body,
        grid=(BB,),
        in_specs=[
            pl.BlockSpec((1, EG, cin), lambda g: (g, 0, 0)),
            pl.BlockSpec((1, SS, cin), lambda g: (g, 0, 0)),
            vec, vec, vec, vec, vec, vec, vec, vec,
            pl.BlockSpec((cin, c), lambda g: (0, 0)),
            pl.BlockSpec((1, c), lambda g: (0, 0)),
            pl.BlockSpec((cin, c), lambda g: (0, 0)),
            pl.BlockSpec((1, c), lambda g: (0, 0)),
        ],
        out_specs=[
            pl.BlockSpec((1, EG, c), lambda g: (g, 0, 0)),
            pl.BlockSpec((1, SS, c), lambda g: (g, 0, 0)),
        ],
        out_shape=[
            jax.ShapeDtypeStruct((BB, EG, c), jnp.float32),
            jax.ShapeDtypeStruct((BB, SS, c), jnp.float32),
        ],
    )(t, u, *st_e, *st_n, we, be, wn, bn)


def _comb_body(t_ref, u_ref, me_ref, se_ref, ge_ref, te_ref,
               mn_ref, sn_ref, gn_ref, tn_ref, h_ref):
    ae = _norm_silu(t_ref[0], me_ref[...], se_ref[...], ge_ref[...],
                    te_ref[...])
    c = ae.shape[1]
    m = jnp.max(ae.reshape(KK, SS, c), axis=0)
    an = _norm_silu(u_ref[0], mn_ref[...], sn_ref[...], gn_ref[...],
                    tn_ref[...])
    h_ref[0] = m + an


def _combine(t, u, st_e, st_n):
    c = t.shape[-1]
    vec = pl.BlockSpec((1, c), lambda g: (0, 0))
    return pl.pallas_call(
        _comb_body,
        grid=(BB,),
        in_specs=[
            pl.BlockSpec((1, EG, c), lambda g: (g, 0, 0)),
            pl.BlockSpec((1, SS, c), lambda g: (g, 0, 0)),
            vec, vec, vec, vec, vec, vec, vec, vec,
        ],
        out_specs=pl.BlockSpec((1, SS, c), lambda g: (g, 0, 0)),
        out_shape=jax.ShapeDtypeStruct((BB, SS, c), jnp.float32),
    )(t, u, *st_e, *st_n)


def _final_body(t_ref, u_ref, me_ref, se_ref, ge_ref, te_ref,
                mn_ref, sn_ref, gn_ref, tn_ref, wo_ref, bo_ref, p_ref):
    g = pl.program_id(0)
    ae = _norm_silu(t_ref[0], me_ref[...], se_ref[...], ge_ref[...],
                    te_ref[...])
    c = ae.shape[1]
    m = jnp.max(ae.reshape(KK, SS, c), axis=0)
    an = _norm_silu(u_ref[0], mn_ref[...], sn_ref[...], gn_ref[...],
                    tn_ref[...])
    h = m + an
    pooled = jnp.sum(h, axis=0, keepdims=True) * (1.0 / SS)
    p_ref[pl.ds(g, 1), :] = _dot(pooled, wo_ref[...]) + bo_ref[...]


def _finalize(t, u, st_e, st_n, w_out, b_out):
    c = t.shape[-1]
    vec = pl.BlockSpec((1, c), lambda g: (0, 0))
    return pl.pallas_call(
        _final_body,
        grid=(BB,),
        in_specs=[
            pl.BlockSpec((1, EG, c), lambda g: (g, 0, 0)),
            pl.BlockSpec((1, SS, c), lambda g: (g, 0, 0)),
            vec, vec, vec, vec, vec, vec, vec, vec,
            pl.BlockSpec((c, 1), lambda g: (0, 0)),
            pl.BlockSpec((1, 1), lambda g: (0, 0)),
        ],
        out_specs=pl.BlockSpec((BB, 1), lambda g: (0, 0)),
        out_shape=jax.ShapeDtypeStruct((BB, 1), jnp.float32),
    )(t, u, *st_e, *st_n, w_out, b_out)


def _bn_stats(arr3d, gamma, beta):
    flat = arr3d.reshape(-1, arr3d.shape[-1])
    mean = jnp.mean(flat, axis=0, keepdims=True)
    var = jnp.var(flat, axis=0, keepdims=True)
    sv = jnp.sqrt(var + EPS)
    return (mean, sv, gamma.reshape(1, -1), beta.reshape(1, -1))


def _dyn_conv(x3d, pos3d, ec, nnp):
    (w1, b1, g1, t1p), (w2, b2, g2, t2p), (w3, b3, g3, t3p) = ec
    (nw1, nb1, ng1, nt1), (nw2, nb2, ng2, nt2), (nw3, nb3, ng3, nt3) = nnp
    t, u = _knn_layer1(pos3d, x3d, w1, b1.reshape(1, -1),
                       nw1, nb1.reshape(1, -1))
    ste, stn = _bn_stats(t, g1, t1p), _bn_stats(u, ng1, nt1)
    t, u = _midlayer(t, u, ste, stn, w2, b2.reshape(1, -1),
                     nw2, nb2.reshape(1, -1))
    ste, stn = _bn_stats(t, g2, t2p), _bn_stats(u, ng2, nt2)
    t, u = _midlayer(t, u, ste, stn, w3, b3.reshape(1, -1),
                     nw3, nb3.reshape(1, -1))
    ste, stn = _bn_stats(t, g3, t3p), _bn_stats(u, ng3, nt3)
    return t, u, ste, stn


def kernel(x, pos, batch, ec1, nn1, ec2, nn2, w_out, b_out):
    del batch
    x3d = x.reshape(BB, SS, -1)
    pos3d = pos.reshape(BB, SS, -1)
    t, u, ste, stn = _dyn_conv(x3d, pos3d, ec1, nn1)
    h1 = _combine(t, u, ste, stn)
    t, u, ste, stn = _dyn_conv(h1, h1, ec2, nn2)
    return _finalize(t, u, ste, stn, w_out, b_out.reshape(1, 1))
